# baseline (speedup 1.0000x reference)
import functools

import jax
import jax.numpy as jnp
from jax import lax
from jax.experimental import pallas as pl
from jax.experimental.pallas import tpu as pltpu
from jax.experimental.pallas import tpu_sc as plsc

_ALPHA = 0.5
_NUM_ITER = 3
_SHRINK = (1.0 - _ALPHA) ** _NUM_ITER

_NUM_WORKERS = 32
_ROWS_PER_STREAM = 128
_ROWS_PER_WORKER = 2 * _ROWS_PER_STREAM


def _sc_update(d):
    mesh = plsc.VectorSubcoreMesh(
        core_axis_name="c", subcore_axis_name="s", num_cores=2, num_subcores=16
    )

    @functools.partial(
        pl.kernel,
        mesh=mesh,
        compiler_params=pltpu.CompilerParams(use_tc_tiling_on_sc=False),
        scratch_types=[
            pltpu.VMEM((2, _ROWS_PER_STREAM), jnp.int32),
            pltpu.VMEM((_ROWS_PER_WORKER, d), jnp.float32),
            pltpu.SemaphoreType.DMA,
            pltpu.SemaphoreType.DMA,
        ],
    )
    def body(x_hbm, idx_hbm, idx_v, rows_v, sem0, sem1):
        wid = lax.axis_index("s") * 2 + lax.axis_index("c")
        pltpu.sync_copy(idx_hbm.at[wid], idx_v)
        g0 = pltpu.async_copy(
            x_hbm.at[idx_v.at[0]], rows_v.at[pl.ds(0, _ROWS_PER_STREAM)], sem0
        )
        g1 = pltpu.async_copy(
            x_hbm.at[idx_v.at[1]],
            rows_v.at[pl.ds(_ROWS_PER_STREAM, _ROWS_PER_STREAM)],
            sem1,
        )
        g0.wait()
        g1.wait()

        nchunks = d // 16

        def unit_update(base, p_count):
            for c in range(nchunks):
                sl = pl.ds(c * 16, 16)
                vals = [rows_v[base + p, sl] for p in range(p_count)]
                acc = vals[0]
                for p in range(1, p_count):
                    acc = acc + vals[p]
                cvec = acc * (1.0 / p_count)
                for p in range(p_count):
                    rows_v[base + p, sl] = cvec + _SHRINK * (vals[p] - cvec)

        @pl.loop(0, 16)
        def _(u):
            unit_update(u * 8, 8)

        @pl.loop(0, 8)
        def _(u):
            unit_update(_ROWS_PER_STREAM + u * 16, 16)

        s0 = pltpu.async_copy(
            rows_v.at[pl.ds(0, _ROWS_PER_STREAM)], x_hbm.at[idx_v.at[0]], sem0
        )
        s1 = pltpu.async_copy(
            rows_v.at[pl.ds(_ROWS_PER_STREAM, _ROWS_PER_STREAM)],
            x_hbm.at[idx_v.at[1]],
            sem1,
        )
        s0.wait()
        s1.wait()

    return body


def kernel(x, groups_a, groups_b):
    B, N, d = x.shape

    boff = (jnp.arange(B, dtype=jnp.int32) * N)[None, :, None]
    ia = (groups_a[:, None, :] + boff).reshape(_NUM_WORKERS, -1)
    ib = (groups_b[:, None, :] + boff).reshape(_NUM_WORKERS, -1)
    idx = jnp.concatenate([ia, ib], axis=1).reshape(_NUM_WORKERS, 2, _ROWS_PER_STREAM)

    xf_ref = jax.new_ref(x.reshape(B * N, d))
    _sc_update(d)(xf_ref, idx)
    return xf_ref[...].reshape(B, N, d)
